# baseline (speedup 1.0000x reference)
import functools

import jax
import jax.numpy as jnp
from jax import lax
from jax.experimental import pallas as pl
from jax.experimental.pallas import tpu as pltpu
from jax.experimental.pallas import tpu_sc as plsc

N = 10000
E = 320000
D = 128
NC = 2
NS = 16
NW = NC * NS
EPT = E // NW
CH = 80
NCHUNK = EPT // CH
N_PAD = 10240
SLAB = N_PAD // NS
L = 16

_mesh = plsc.VectorSubcoreMesh(core_axis_name="c", subcore_axis_name="s",
                               num_cores=NC, num_subcores=NS)


def _deg_body(dst_hbm, w_hbm, out_hbm, dst_v, w_v, acc_v):
    c = lax.axis_index("c")
    s = lax.axis_index("s")
    wid = c * NS + s

    def _zero(i, _):
        acc_v[pl.ds(i * L, L)] = jnp.zeros((L,), jnp.float32)
        return 0
    lax.fori_loop(0, N_PAD // L, _zero, 0)

    base = wid * EPT
    pltpu.sync_copy(dst_hbm.at[pl.ds(base, EPT)], dst_v)
    pltpu.sync_copy(w_hbm.at[pl.ds(base, EPT)], w_v)

    def _accum(g, _):
        idx = dst_v[pl.ds(g * L, L)]
        vals = w_v[pl.ds(g * L, L)]
        plsc.addupdate_scatter(acc_v, [idx], vals)
        return 0
    lax.fori_loop(0, EPT // L, _accum, 0)

    pltpu.sync_copy(acc_v, out_hbm.at[c, s])


def _deg_partials(dst, w):
    return pl.kernel(
        _deg_body,
        out_type=jax.ShapeDtypeStruct((NC, NS, N_PAD), jnp.float32),
        mesh=_mesh,
        compiler_params=pltpu.CompilerParams(needs_layout_passes=False),
        scratch_types=[
            pltpu.VMEM((EPT,), jnp.int32),
            pltpu.VMEM((EPT,), jnp.float32),
            pltpu.VMEM((N_PAD,), jnp.float32),
        ],
    )(dst, w)


NBUF = 4
_BCAST_DNUMS = lax.GatherDimensionNumbers(
    offset_dims=(), collapsed_slice_dims=(0,), start_index_map=(0,))


def _bcast_lane(vec, e):
    idx = jnp.full((L,), e, jnp.int32)
    return lax.gather(vec, idx[:, None], _BCAST_DNUMS, slice_sizes=(1,),
                      mode=lax.GatherScatterMode.PROMISE_IN_BOUNDS)


def _msg_body(y_hbm, iw_hbm, dst_hbm, out_hbm,
              iw_bufs, dst_bufs, g_bufs, s_bufs,
              isem, dsem, gsem, ssem, acc_sh):
    c = lax.axis_index("c")
    s = lax.axis_index("s")
    wid = c * NS + s
    tc0 = wid * NCHUNK

    def _iw_copy(b, p, t):
        return pltpu.make_async_copy(iw_hbm.at[tc0 + t], iw_bufs[b][p],
                                     isem.at[b, p])

    def _dst_copy(b, p, t):
        return pltpu.make_async_copy(dst_hbm.at[tc0 + t], dst_bufs[b][p],
                                     dsem.at[b, p])

    def _gather(b, p):
        return pltpu.make_async_copy(y_hbm.at[iw_bufs[b][p].at[0]],
                                     g_bufs[b], gsem.at[b])

    def _scatter(b, p):
        return pltpu.make_async_copy(s_bufs[b], acc_sh.at[dst_bufs[b][p]],
                                     ssem.at[b])

    def _scale(b, p):
        @plsc.parallel_loop(0, CH // L, unroll=CH // L)
        def _grp(g):
            w16 = plsc.bitcast(iw_bufs[b][p][1, pl.ds(g * L, L)], jnp.float32)
            for e in range(L):
                row = g * L + e
                wb = _bcast_lane(w16, e)
                for j in range(D // L):
                    sl = pl.ds(j * L, L)
                    s_bufs[b][row, sl] = g_bufs[b][row, sl] * wb

    def _turn(b, p, t):
        _gather(b, p).wait()

        @pl.when(t >= 2)
        def _():
            _scatter(b, p).wait()

        _scale(b, p)
        _scatter(b, p).start(add=True)

        @pl.when(t + 4 < NCHUNK)
        def _():
            _iw_copy(b, p, t + 4).start()

        @pl.when(t + 2 < NCHUNK)
        def _():
            _dst_copy(b, 1 - p, t + 2).start()
            _iw_copy(b, 1 - p, t + 2).wait()
            _gather(b, 1 - p).start()
            _dst_copy(b, 1 - p, t + 2).wait()

    for b in range(2):
        _iw_copy(b, 0, b).start()
        _iw_copy(b, 1, b + 2).start()
        _dst_copy(b, 0, b).start()

    @plsc.parallel_loop(0, CH, unroll=8)
    def _zrow(i):
        for j in range(D // L):
            s_bufs[0][i, pl.ds(j * L, L)] = jnp.zeros((L,), jnp.float32)

    for k in range(SLAB // CH):
        pltpu.make_async_copy(
            s_bufs[0], acc_sh.at[pl.ds(s * SLAB + k * CH, CH)],
            ssem.at[0]).start()
    for k in range(SLAB // CH):
        pltpu.make_async_copy(
            s_bufs[0], acc_sh.at[pl.ds(s * SLAB + k * CH, CH)],
            ssem.at[0]).wait()
    plsc.subcore_barrier()

    for b in range(2):
        _iw_copy(b, 0, b).wait()
        _gather(b, 0).start()
        _dst_copy(b, 0, b).wait()

    def _block(blk, _):
        t0 = blk * 4
        _turn(0, 0, t0)
        _turn(1, 0, t0 + 1)
        _turn(0, 1, t0 + 2)
        _turn(1, 1, t0 + 3)
        return 0
    lax.fori_loop(0, NCHUNK // 4, _block, 0)

    _turn(0, 0, NCHUNK - 1)
    _scatter(0, 0).wait()
    _scatter(1, 1).wait()

    plsc.subcore_barrier()
    pltpu.sync_copy(acc_sh.at[pl.ds(s * SLAB, SLAB)],
                    out_hbm.at[c, pl.ds(s * SLAB, SLAB)])


def _msg_pass(y, iw_packed, dst2):
    return pl.kernel(
        _msg_body,
        out_type=jax.ShapeDtypeStruct((NC, N_PAD, D), jnp.float32),
        mesh=_mesh,
        compiler_params=pltpu.CompilerParams(needs_layout_passes=False),
        scratch_types=[
            [[pltpu.VMEM((2, CH), jnp.int32) for _ in range(2)]
             for _ in range(2)],
            [[pltpu.VMEM((CH,), jnp.int32) for _ in range(2)]
             for _ in range(2)],
            [pltpu.VMEM((CH, D), jnp.float32) for _ in range(2)],
            [pltpu.VMEM((CH, D), jnp.float32) for _ in range(2)],
            pltpu.SemaphoreType.DMA((2, 2)),
            pltpu.SemaphoreType.DMA((2, 2)),
            pltpu.SemaphoreType.DMA((2,)),
            pltpu.SemaphoreType.DMA((2,)),
            pltpu.VMEM_SHARED((N_PAD, D), jnp.float32),
        ],
    )(y, iw_packed, dst2)


R = 1000
G = N // R


def _dis_body(degp_ref, dis_ref):
    deg = 1.0 + jnp.sum(degp_ref[...], axis=0, keepdims=True)
    dis_ref[...] = lax.rsqrt(deg)


def _dis_kernel(degp):
    return pl.pallas_call(
        _dis_body,
        out_shape=jax.ShapeDtypeStruct((1, N_PAD), jnp.float32),
    )(degp)


def _y1_body(x_ref, w_ref, dis_ref, y_ref):
    xw = jnp.dot(x_ref[...], w_ref[...], preferred_element_type=jnp.float32)
    y_ref[...] = dis_ref[...] * xw


def _y1_kernel(x, W1, dis_col):
    return pl.pallas_call(
        _y1_body,
        grid=(G,),
        in_specs=[
            pl.BlockSpec((R, D), lambda i: (i, 0)),
            pl.BlockSpec((D, D), lambda i: (0, 0)),
            pl.BlockSpec((R, 1), lambda i: (i, 0)),
        ],
        out_specs=pl.BlockSpec((R, D), lambda i: (i, 0)),
        out_shape=jax.ShapeDtypeStruct((N, D), jnp.float32),
    )(x, W1, dis_col)


def _mid_body(acc_ref, y_ref, dis_ref, b_ref, w2_ref, out_ref):
    a = acc_ref[0] + acc_ref[1]
    h = jnp.maximum(dis_ref[...] * (a + y_ref[...]) + b_ref[...], 0.0)
    hw = jnp.dot(h, w2_ref[...], preferred_element_type=jnp.float32)
    out_ref[...] = dis_ref[...] * hw


def _mid_kernel(acc1, y1, dis_col, b1, W2):
    return pl.pallas_call(
        _mid_body,
        grid=(G,),
        in_specs=[
            pl.BlockSpec((NC, R, D), lambda i: (0, i, 0)),
            pl.BlockSpec((R, D), lambda i: (i, 0)),
            pl.BlockSpec((R, 1), lambda i: (i, 0)),
            pl.BlockSpec((1, D), lambda i: (0, 0)),
            pl.BlockSpec((D, D), lambda i: (0, 0)),
        ],
        out_specs=pl.BlockSpec((R, D), lambda i: (i, 0)),
        out_shape=jax.ShapeDtypeStruct((N, D), jnp.float32),
    )(acc1, y1, dis_col, b1, W2)


def _fin_body(acc_ref, y_ref, dis_ref, b_ref, wm_ref, bm_ref, out_ref,
              sum_ref):
    i = pl.program_id(0)
    a = acc_ref[0] + acc_ref[1]
    h = jnp.maximum(dis_ref[...] * (a + y_ref[...]) + b_ref[...], 0.0)
    part = jnp.sum(h, axis=0, keepdims=True)

    @pl.when(i == 0)
    def _():
        sum_ref[...] = part

    @pl.when(i > 0)
    def _():
        sum_ref[...] = sum_ref[...] + part

    @pl.when(i == G - 1)
    def _():
        out_ref[...] = (
            jnp.dot(sum_ref[...] * (1.0 / N), wm_ref[...],
                    preferred_element_type=jnp.float32) + bm_ref[...]
        )


def _fin_kernel(acc2, y2, dis_col, b2, Wm, bm):
    return pl.pallas_call(
        _fin_body,
        grid=(G,),
        in_specs=[
            pl.BlockSpec((NC, R, D), lambda i: (0, i, 0)),
            pl.BlockSpec((R, D), lambda i: (i, 0)),
            pl.BlockSpec((R, 1), lambda i: (i, 0)),
            pl.BlockSpec((1, D), lambda i: (0, 0)),
            pl.BlockSpec((D, D), lambda i: (0, 0)),
            pl.BlockSpec((1, D), lambda i: (0, 0)),
        ],
        out_specs=pl.BlockSpec((1, D), lambda i: (0, 0)),
        out_shape=jax.ShapeDtypeStruct((1, D), jnp.float32),
        scratch_shapes=[pltpu.VMEM((1, D), jnp.float32)],
    )(acc2, y2, dis_col, b2, Wm, bm)


def kernel(x, edge_obj_to_obj, edge_weight, W1, b1, W2, b2, Wm, bm):
    src = edge_obj_to_obj[0]
    dst = edge_obj_to_obj[1]

    degp = _deg_partials(dst, edge_weight)
    dis = _dis_kernel(degp.reshape(NC * NS, N_PAD))
    dis_col = dis.reshape(N_PAD, 1)[:N]

    wbits = lax.bitcast_convert_type(edge_weight, jnp.int32)
    iw_packed = jnp.stack(
        [src.reshape(E // CH, CH), wbits.reshape(E // CH, CH)], axis=1)
    dst2 = dst.reshape(E // CH, CH)
    y1 = _y1_kernel(x, W1, dis_col)
    acc1 = _msg_pass(y1, iw_packed, dst2)
    y2 = _mid_kernel(acc1, y1, dis_col, b1.reshape(1, D), W2)
    acc2 = _msg_pass(y2, iw_packed, dst2)
    g = _fin_kernel(acc2, y2, dis_col, b2.reshape(1, D), Wm,
                    bm.reshape(1, D))
    return g


# speedup vs baseline: 1.3015x; 1.3015x over previous
import functools

import jax
import jax.numpy as jnp
from jax import lax
from jax.experimental import pallas as pl
from jax.experimental.pallas import tpu as pltpu
from jax.experimental.pallas import tpu_sc as plsc

N = 10000
E = 320000
D = 128
NC = 2
NS = 16
NW = NC * NS
EPT = E // NW
CH = 80
NCHUNK = EPT // CH
N_PAD = 10240
SLAB = N_PAD // NS
L = 16

_mesh = plsc.VectorSubcoreMesh(core_axis_name="c", subcore_axis_name="s",
                               num_cores=NC, num_subcores=NS)


def _deg_body(dst_hbm, w_hbm, out_hbm, dst_v, w_v, acc_v):
    c = lax.axis_index("c")
    s = lax.axis_index("s")
    wid = c * NS + s

    def _zero(i, _):
        acc_v[pl.ds(i * L, L)] = jnp.zeros((L,), jnp.float32)
        return 0
    lax.fori_loop(0, N_PAD // L, _zero, 0)

    base = wid * EPT
    pltpu.sync_copy(dst_hbm.at[pl.ds(base, EPT)], dst_v)
    pltpu.sync_copy(w_hbm.at[pl.ds(base, EPT)], w_v)

    def _accum(g, _):
        idx = dst_v[pl.ds(g * L, L)]
        vals = w_v[pl.ds(g * L, L)]
        plsc.addupdate_scatter(acc_v, [idx], vals)
        return 0
    lax.fori_loop(0, EPT // L, _accum, 0)

    pltpu.sync_copy(acc_v, out_hbm.at[c, s])


def _deg_partials(dst, w):
    return pl.kernel(
        _deg_body,
        out_type=jax.ShapeDtypeStruct((NC, NS, N_PAD), jnp.float32),
        mesh=_mesh,
        compiler_params=pltpu.CompilerParams(needs_layout_passes=False),
        scratch_types=[
            pltpu.VMEM((EPT,), jnp.int32),
            pltpu.VMEM((EPT,), jnp.float32),
            pltpu.VMEM((N_PAD,), jnp.float32),
        ],
    )(dst, w)


NBUF = 4
_BCAST_DNUMS = lax.GatherDimensionNumbers(
    offset_dims=(), collapsed_slice_dims=(0,), start_index_map=(0,))


def _bcast_lane(vec, e):
    idx = jnp.full((L,), e, jnp.int32)
    return lax.gather(vec, idx[:, None], _BCAST_DNUMS, slice_sizes=(1,),
                      mode=lax.GatherScatterMode.PROMISE_IN_BOUNDS)


def _msg_body(y_hbm, iw_hbm, dst_hbm, out_hbm,
              iw_bufs, dst_bufs, g_bufs,
              isem, dsem, gsem, ssem, acc_sh):
    c = lax.axis_index("c")
    s = lax.axis_index("s")
    wid = c * NS + s
    tc0 = wid * NCHUNK

    def _iw_copy(b, p, t):
        return pltpu.make_async_copy(iw_hbm.at[tc0 + t], iw_bufs[b][p],
                                     isem.at[b, p])

    def _dst_copy(b, p, t):
        return pltpu.make_async_copy(dst_hbm.at[tc0 + t], dst_bufs[b][p],
                                     dsem.at[b, p])

    def _gather(b, p):
        return pltpu.make_async_copy(y_hbm.at[iw_bufs[b][p].at[0]],
                                     g_bufs[b][p], gsem.at[b])

    def _scatter(b, p):
        return pltpu.make_async_copy(g_bufs[b][p], acc_sh.at[dst_bufs[b][p]],
                                     ssem.at[b])

    def _scale(b, p):
        @plsc.parallel_loop(0, CH // L, unroll=CH // L)
        def _grp(g):
            w16 = plsc.bitcast(iw_bufs[b][p][1, pl.ds(g * L, L)], jnp.float32)
            for e in range(L):
                row = g * L + e
                wb = _bcast_lane(w16, e)
                for j in range(D // L):
                    sl = pl.ds(j * L, L)
                    g_bufs[b][p][row, sl] = g_bufs[b][p][row, sl] * wb

    def _turn(b, p, t):
        _gather(b, p).wait()

        @pl.when(t >= 2)
        def _():
            _scatter(b, 1 - p).wait()

        @pl.when(t + 2 < NCHUNK)
        def _():
            _iw_copy(b, 1 - p, t + 2).wait()
            _gather(b, 1 - p).start()
            _dst_copy(b, 1 - p, t + 2).start()

        _scale(b, p)
        _dst_copy(b, p, t).wait()
        _scatter(b, p).start(add=True)

        @pl.when(t + 4 < NCHUNK)
        def _():
            _iw_copy(b, p, t + 4).start()

    for b in range(2):
        _iw_copy(b, 0, b).start()
        _iw_copy(b, 1, b + 2).start()
        _dst_copy(b, 0, b).start()

    @plsc.parallel_loop(0, CH, unroll=8)
    def _zrow(i):
        for j in range(D // L):
            g_bufs[0][0][i, pl.ds(j * L, L)] = jnp.zeros((L,), jnp.float32)

    for k in range(SLAB // CH):
        pltpu.make_async_copy(
            g_bufs[0][0], acc_sh.at[pl.ds(s * SLAB + k * CH, CH)],
            ssem.at[0]).start()
    for k in range(SLAB // CH):
        pltpu.make_async_copy(
            g_bufs[0][0], acc_sh.at[pl.ds(s * SLAB + k * CH, CH)],
            ssem.at[0]).wait()
    plsc.subcore_barrier()

    for b in range(2):
        _iw_copy(b, 0, b).wait()
        _gather(b, 0).start()

    def _block(blk, _):
        t0 = blk * 4
        _turn(0, 0, t0)
        _turn(1, 0, t0 + 1)
        _turn(0, 1, t0 + 2)
        _turn(1, 1, t0 + 3)
        return 0
    lax.fori_loop(0, NCHUNK // 4, _block, 0)

    _turn(0, 0, NCHUNK - 1)
    _scatter(0, 0).wait()
    _scatter(1, 1).wait()

    plsc.subcore_barrier()
    pltpu.sync_copy(acc_sh.at[pl.ds(s * SLAB, SLAB)],
                    out_hbm.at[c, pl.ds(s * SLAB, SLAB)])


def _msg_pass(y, iw_packed, dst2):
    return pl.kernel(
        _msg_body,
        out_type=jax.ShapeDtypeStruct((NC, N_PAD, D), jnp.float32),
        mesh=_mesh,
        compiler_params=pltpu.CompilerParams(needs_layout_passes=False),
        scratch_types=[
            [[pltpu.VMEM((2, CH), jnp.int32) for _ in range(2)]
             for _ in range(2)],
            [[pltpu.VMEM((CH,), jnp.int32) for _ in range(2)]
             for _ in range(2)],
            [[pltpu.VMEM((CH, D), jnp.float32) for _ in range(2)]
             for _ in range(2)],
            pltpu.SemaphoreType.DMA((2, 2)),
            pltpu.SemaphoreType.DMA((2, 2)),
            pltpu.SemaphoreType.DMA((2,)),
            pltpu.SemaphoreType.DMA((2,)),
            pltpu.VMEM_SHARED((N_PAD, D), jnp.float32),
        ],
    )(y, iw_packed, dst2)


R = 1000
G = N // R


def _dis_body(degp_ref, dis_ref):
    deg = 1.0 + jnp.sum(degp_ref[...], axis=0, keepdims=True)
    dis_ref[...] = lax.rsqrt(deg)


def _dis_kernel(degp):
    return pl.pallas_call(
        _dis_body,
        out_shape=jax.ShapeDtypeStruct((1, N_PAD), jnp.float32),
    )(degp)


def _y1_body(x_ref, w_ref, dis_ref, y_ref):
    xw = jnp.dot(x_ref[...], w_ref[...], preferred_element_type=jnp.float32)
    y_ref[...] = dis_ref[...] * xw


def _y1_kernel(x, W1, dis_col):
    return pl.pallas_call(
        _y1_body,
        grid=(G,),
        in_specs=[
            pl.BlockSpec((R, D), lambda i: (i, 0)),
            pl.BlockSpec((D, D), lambda i: (0, 0)),
            pl.BlockSpec((R, 1), lambda i: (i, 0)),
        ],
        out_specs=pl.BlockSpec((R, D), lambda i: (i, 0)),
        out_shape=jax.ShapeDtypeStruct((N, D), jnp.float32),
    )(x, W1, dis_col)


def _mid_body(acc_ref, y_ref, dis_ref, b_ref, w2_ref, out_ref):
    a = acc_ref[0] + acc_ref[1]
    h = jnp.maximum(dis_ref[...] * (a + y_ref[...]) + b_ref[...], 0.0)
    hw = jnp.dot(h, w2_ref[...], preferred_element_type=jnp.float32)
    out_ref[...] = dis_ref[...] * hw


def _mid_kernel(acc1, y1, dis_col, b1, W2):
    return pl.pallas_call(
        _mid_body,
        grid=(G,),
        in_specs=[
            pl.BlockSpec((NC, R, D), lambda i: (0, i, 0)),
            pl.BlockSpec((R, D), lambda i: (i, 0)),
            pl.BlockSpec((R, 1), lambda i: (i, 0)),
            pl.BlockSpec((1, D), lambda i: (0, 0)),
            pl.BlockSpec((D, D), lambda i: (0, 0)),
        ],
        out_specs=pl.BlockSpec((R, D), lambda i: (i, 0)),
        out_shape=jax.ShapeDtypeStruct((N, D), jnp.float32),
    )(acc1, y1, dis_col, b1, W2)


def _fin_body(acc_ref, y_ref, dis_ref, b_ref, wm_ref, bm_ref, out_ref,
              sum_ref):
    i = pl.program_id(0)
    a = acc_ref[0] + acc_ref[1]
    h = jnp.maximum(dis_ref[...] * (a + y_ref[...]) + b_ref[...], 0.0)
    part = jnp.sum(h, axis=0, keepdims=True)

    @pl.when(i == 0)
    def _():
        sum_ref[...] = part

    @pl.when(i > 0)
    def _():
        sum_ref[...] = sum_ref[...] + part

    @pl.when(i == G - 1)
    def _():
        out_ref[...] = (
            jnp.dot(sum_ref[...] * (1.0 / N), wm_ref[...],
                    preferred_element_type=jnp.float32) + bm_ref[...]
        )


def _fin_kernel(acc2, y2, dis_col, b2, Wm, bm):
    return pl.pallas_call(
        _fin_body,
        grid=(G,),
        in_specs=[
            pl.BlockSpec((NC, R, D), lambda i: (0, i, 0)),
            pl.BlockSpec((R, D), lambda i: (i, 0)),
            pl.BlockSpec((R, 1), lambda i: (i, 0)),
            pl.BlockSpec((1, D), lambda i: (0, 0)),
            pl.BlockSpec((D, D), lambda i: (0, 0)),
            pl.BlockSpec((1, D), lambda i: (0, 0)),
        ],
        out_specs=pl.BlockSpec((1, D), lambda i: (0, 0)),
        out_shape=jax.ShapeDtypeStruct((1, D), jnp.float32),
        scratch_shapes=[pltpu.VMEM((1, D), jnp.float32)],
    )(acc2, y2, dis_col, b2, Wm, bm)


def kernel(x, edge_obj_to_obj, edge_weight, W1, b1, W2, b2, Wm, bm):
    src = edge_obj_to_obj[0]
    dst = edge_obj_to_obj[1]

    degp = _deg_partials(dst, edge_weight)
    dis = _dis_kernel(degp.reshape(NC * NS, N_PAD))
    dis_col = dis.reshape(N_PAD, 1)[:N]

    wbits = lax.bitcast_convert_type(edge_weight, jnp.int32)
    iw_packed = jnp.stack(
        [src.reshape(E // CH, CH), wbits.reshape(E // CH, CH)], axis=1)
    dst2 = dst.reshape(E // CH, CH)
    y1 = _y1_kernel(x, W1, dis_col)
    acc1 = _msg_pass(y1, iw_packed, dst2)
    y2 = _mid_kernel(acc1, y1, dis_col, b1.reshape(1, D), W2)
    acc2 = _msg_pass(y2, iw_packed, dst2)
    g = _fin_kernel(acc2, y2, dis_col, b2.reshape(1, D), Wm,
                    bm.reshape(1, D))
    return g
